# Initial kernel scaffold; baseline (speedup 1.0000x reference)
import jax
import jax.numpy as jnp
from jax.experimental import pallas as pl


def kernel(prediction, target, weights, thresholds):
    raise NotImplementedError("write your pallas kernel here")


# speedup vs baseline: 3079.8400x; 3079.8400x over previous
import functools

import jax
import jax.numpy as jnp
from jax import lax
from jax.experimental import pallas as pl
from jax.experimental.pallas import tpu as pltpu
from jax.experimental.pallas import tpu_sc as plsc

_L = 16
_CHUNK = 16384


def _make_sc_wmse(n_total):
    info = plsc.get_sparse_core_info()
    nc, ns = info.num_cores, info.num_subcores
    nw = nc * ns
    assert n_total % (nw * 2 * _CHUNK) == 0
    per_w = n_total // nw
    steps = per_w // _CHUNK

    mesh = plsc.VectorSubcoreMesh(core_axis_name="c", subcore_axis_name="s")

    @functools.partial(
        pl.kernel,
        mesh=mesh,
        out_type=jax.ShapeDtypeStruct((nw, _L), jnp.float32),
        scratch_types=[
            pltpu.VMEM((_CHUNK,), jnp.float32),
            pltpu.VMEM((_CHUNK,), jnp.float32),
            pltpu.VMEM((_CHUNK,), jnp.float32),
            pltpu.VMEM((_CHUNK,), jnp.float32),
            pltpu.VMEM((4 * _L,), jnp.float32),
            pltpu.VMEM((5 * _L,), jnp.float32),
            pltpu.VMEM((_L,), jnp.float32),
            pltpu.SemaphoreType.DMA,
            pltpu.SemaphoreType.DMA,
            pltpu.SemaphoreType.DMA,
            pltpu.SemaphoreType.DMA,
        ],
    )
    def wmse(pred_hbm, tgt_hbm, thr_hbm, dw_hbm, out_hbm,
             p0, p1, t0, t1, thr_v, dw_v, acc_v,
             sp0, sp1, st0, st1):
        wid = lax.axis_index("s") * nc + lax.axis_index("c")
        base = wid * per_w
        pbufs, tbufs = (p0, p1), (t0, t1)
        psems, tsems = (sp0, sp1), (st0, st1)

        pltpu.sync_copy(thr_hbm, thr_v)
        pltpu.sync_copy(dw_hbm, dw_v)

        def dma_start(slot, g):
            off = base + g * _CHUNK
            pltpu.async_copy(pred_hbm.at[pl.ds(off, _CHUNK)], pbufs[slot],
                             psems[slot])
            pltpu.async_copy(tgt_hbm.at[pl.ds(off, _CHUNK)], tbufs[slot],
                             tsems[slot])

        def dma_wait(slot):
            pltpu.make_async_copy(pred_hbm.at[pl.ds(0, _CHUNK)], pbufs[slot],
                                  psems[slot]).wait()
            pltpu.make_async_copy(tgt_hbm.at[pl.ds(0, _CHUNK)], tbufs[slot],
                                  tsems[slot]).wait()

        dma_start(0, 0)
        dma_start(1, 1)

        th0 = thr_v[pl.ds(0 * _L, _L)]
        th1 = thr_v[pl.ds(1 * _L, _L)]
        th2 = thr_v[pl.ds(2 * _L, _L)]
        th3 = thr_v[pl.ds(3 * _L, _L)]
        dw0 = dw_v[pl.ds(0 * _L, _L)]
        dw1 = dw_v[pl.ds(1 * _L, _L)]
        dw2 = dw_v[pl.ds(2 * _L, _L)]
        dw3 = dw_v[pl.ds(3 * _L, _L)]
        dw4 = dw_v[pl.ds(4 * _L, _L)]
        zero = jnp.zeros((_L,), jnp.float32)

        def chunk_acc(pbuf, tbuf, acc):
            def body(i, a):
                p = pbuf[pl.ds(i * _L, _L)]
                t = tbuf[pl.ds(i * _L, _L)]
                w = dw0
                w = w + jnp.where(t >= th0, dw1, zero)
                w = w + jnp.where(t >= th1, dw2, zero)
                w = w + jnp.where(t >= th2, dw3, zero)
                w = w + jnp.where(t >= th3, dw4, zero)
                d = p - t
                return a + w * (d * d)
            return lax.fori_loop(0, _CHUNK // _L, body, acc)

        def outer(k, acc):
            for b in range(2):
                g = 2 * k + b
                dma_wait(b)
                acc = chunk_acc(pbufs[b], tbufs[b], acc)

                @pl.when(g + 2 < steps)
                def _():
                    dma_start(b, g + 2)
            return acc

        acc = lax.fori_loop(0, steps // 2, outer, zero)
        acc_v[...] = acc
        pltpu.sync_copy(acc_v, out_hbm.at[wid])

    return wmse, nw


def kernel(prediction, target, weights, thresholds):
    n = prediction.size
    pred = prediction.reshape(n)
    tgt = target.reshape(n)
    dw = jnp.concatenate([weights[:1], jnp.diff(weights)])
    thr_b = jnp.broadcast_to(thresholds[:, None], (4, _L)).reshape(4 * _L)
    dw_b = jnp.broadcast_to(dw[:, None], (5, _L)).reshape(5 * _L)

    sc_wmse, _ = _make_sc_wmse(n)
    partials = sc_wmse(pred, tgt, thr_b, dw_b)
    return jnp.sum(partials) / n


# speedup vs baseline: 11520.8808x; 3.7407x over previous
import functools

import jax
import jax.numpy as jnp
from jax import lax
from jax.experimental import pallas as pl
from jax.experimental.pallas import tpu as pltpu
from jax.experimental.pallas import tpu_sc as plsc

_L = 16
_CHUNK_ROWS = 16


def _make_sc_wmse(row_offset, sc_rows, cols):
    info = plsc.get_sparse_core_info()
    nc, ns = info.num_cores, info.num_subcores
    nw = nc * ns
    assert sc_rows % (nw * 2 * _CHUNK_ROWS) == 0
    per_w = sc_rows // nw
    steps = per_w // _CHUNK_ROWS
    vregs_per_row = cols // _L

    mesh = plsc.VectorSubcoreMesh(core_axis_name="c", subcore_axis_name="s")

    @functools.partial(
        pl.kernel,
        mesh=mesh,
        out_type=jax.ShapeDtypeStruct((nw, _L), jnp.float32),
        scratch_types=[
            pltpu.VMEM((_CHUNK_ROWS, cols), jnp.float32),
            pltpu.VMEM((_CHUNK_ROWS, cols), jnp.float32),
            pltpu.VMEM((_CHUNK_ROWS, cols), jnp.float32),
            pltpu.VMEM((_CHUNK_ROWS, cols), jnp.float32),
            pltpu.VMEM((4 * _L,), jnp.float32),
            pltpu.VMEM((5 * _L,), jnp.float32),
            pltpu.VMEM((_L,), jnp.float32),
            pltpu.SemaphoreType.DMA,
            pltpu.SemaphoreType.DMA,
            pltpu.SemaphoreType.DMA,
            pltpu.SemaphoreType.DMA,
        ],
    )
    def wmse(pred_hbm, tgt_hbm, thr_hbm, dw_hbm, out_hbm,
             p0, p1, t0, t1, thr_v, dw_v, acc_v,
             sp0, sp1, st0, st1):
        wid = lax.axis_index("s") * nc + lax.axis_index("c")
        base = row_offset + wid * per_w
        pbufs, tbufs = (p0, p1), (t0, t1)
        psems, tsems = (sp0, sp1), (st0, st1)

        pltpu.sync_copy(thr_hbm, thr_v)
        pltpu.sync_copy(dw_hbm, dw_v)

        def dma_start(slot, g):
            row = base + g * _CHUNK_ROWS
            pltpu.async_copy(pred_hbm.at[pl.ds(row, _CHUNK_ROWS)],
                             pbufs[slot], psems[slot])
            pltpu.async_copy(tgt_hbm.at[pl.ds(row, _CHUNK_ROWS)],
                             tbufs[slot], tsems[slot])

        def dma_wait(slot):
            pltpu.make_async_copy(pred_hbm.at[pl.ds(0, _CHUNK_ROWS)],
                                  pbufs[slot], psems[slot]).wait()
            pltpu.make_async_copy(tgt_hbm.at[pl.ds(0, _CHUNK_ROWS)],
                                  tbufs[slot], tsems[slot]).wait()

        dma_start(0, 0)
        dma_start(1, 1)

        ths = [thr_v[pl.ds(k * _L, _L)] for k in range(4)]
        dws = [dw_v[pl.ds(k * _L, _L)] for k in range(5)]
        zero = jnp.zeros((_L,), jnp.float32)
        unroll = 4

        def chunk_acc(pbuf, tbuf, accs):
            def row_body(r, accs):
                for c in range(0, vregs_per_row, unroll):
                    ps = [pbuf[r, pl.ds((c + j) * _L, _L)]
                          for j in range(unroll)]
                    ts = [tbuf[r, pl.ds((c + j) * _L, _L)]
                          for j in range(unroll)]
                    d2s = [(p - t) * (p - t) for p, t in zip(ps, ts)]
                    new = []
                    for k in range(5):
                        if k == 0:
                            terms = d2s
                        else:
                            terms = [jnp.where(t >= ths[k - 1], d2, zero)
                                     for t, d2 in zip(ts, d2s)]
                        s01 = terms[0] + terms[1]
                        s23 = terms[2] + terms[3]
                        new.append(accs[k] + (s01 + s23))
                    accs = tuple(new)
                return accs
            return lax.fori_loop(0, _CHUNK_ROWS, row_body, accs)

        def outer(k, accs):
            for b in range(2):
                g = 2 * k + b
                dma_wait(b)
                accs = chunk_acc(pbufs[b], tbufs[b], accs)

                @pl.when(g + 2 < steps)
                def _():
                    dma_start(b, g + 2)
            return accs

        accs = lax.fori_loop(0, steps // 2, outer, (zero,) * 5)
        acc = dws[0] * accs[0]
        for k in range(1, 5):
            acc = acc + dws[k] * accs[k]
        acc_v[...] = acc
        pltpu.sync_copy(acc_v, out_hbm.at[wid])

    return wmse


_TC_BLOCK_ROWS = 512


def _tc_body(pred_ref, tgt_ref, thr_ref, dw_ref, out_ref, acc_ref):
    i = pl.program_id(0)

    @pl.when(i == 0)
    def _():
        acc_ref[...] = jnp.zeros_like(acc_ref)

    p = pred_ref[...]
    t = tgt_ref[...]
    d2 = (p - t) * (p - t)
    w = jnp.full_like(t, dw_ref[0])
    for k in range(4):
        w = w + jnp.where(t >= thr_ref[k], dw_ref[k + 1], 0.0)
    acc_ref[...] += w * d2

    @pl.when(i == pl.num_programs(0) - 1)
    def _():
        out_ref[0] = jnp.sum(acc_ref[...])


def _tc_wmse(pred2d, tgt2d, thr, dw, tc_rows):
    cols = pred2d.shape[1]
    grid = tc_rows // _TC_BLOCK_ROWS
    blk = pl.BlockSpec((_TC_BLOCK_ROWS, cols), lambda i: (i, 0))
    smem = pl.BlockSpec(memory_space=pltpu.SMEM)
    out = pl.pallas_call(
        _tc_body,
        grid=(grid,),
        in_specs=[blk, blk, smem, smem],
        out_specs=pl.BlockSpec(memory_space=pltpu.SMEM),
        out_shape=jax.ShapeDtypeStruct((1,), jnp.float32),
        scratch_shapes=[pltpu.VMEM((_TC_BLOCK_ROWS, cols), jnp.float32)],
    )(pred2d, tgt2d, thr, dw)
    return out[0]


_SC_ROWS = 8192


def kernel(prediction, target, weights, thresholds):
    n = prediction.size
    cols = prediction.shape[-1]
    rows = n // cols
    pred2d = prediction.reshape(rows, cols)
    tgt2d = target.reshape(rows, cols)
    dw = jnp.concatenate([weights[:1], jnp.diff(weights)])

    sc_rows = min(_SC_ROWS, rows)
    tc_rows = rows - sc_rows
    total = jnp.float32(0.0)
    if sc_rows > 0:
        thr_b = jnp.broadcast_to(thresholds[:, None], (4, _L)).reshape(4 * _L)
        dw_b = jnp.broadcast_to(dw[:, None], (5, _L)).reshape(5 * _L)
        sc_wmse = _make_sc_wmse(tc_rows, sc_rows, cols)
        partials = sc_wmse(pred2d, tgt2d, thr_b, dw_b)
        total = total + jnp.sum(partials)
    if tc_rows > 0:
        total = total + _tc_wmse(pred2d, tgt2d, thresholds, dw, tc_rows)
    return total / n


# speedup vs baseline: 11760.7435x; 1.0208x over previous
import functools

import jax
import jax.numpy as jnp
from jax import lax
from jax.experimental import pallas as pl
from jax.experimental.pallas import tpu as pltpu
from jax.experimental.pallas import tpu_sc as plsc

_L = 16
_CHUNK_ROWS = 16


def _make_sc_wmse(row_offset, sc_rows, cols):
    info = plsc.get_sparse_core_info()
    nc, ns = info.num_cores, info.num_subcores
    nw = nc * ns
    assert sc_rows % (nw * 2 * _CHUNK_ROWS) == 0
    per_w = sc_rows // nw
    steps = per_w // _CHUNK_ROWS
    vregs_per_row = cols // _L

    mesh = plsc.VectorSubcoreMesh(core_axis_name="c", subcore_axis_name="s")

    @functools.partial(
        pl.kernel,
        mesh=mesh,
        out_type=jax.ShapeDtypeStruct((nw, _L), jnp.float32),
        scratch_types=[
            pltpu.VMEM((_CHUNK_ROWS, cols), jnp.float32),
            pltpu.VMEM((_CHUNK_ROWS, cols), jnp.float32),
            pltpu.VMEM((_CHUNK_ROWS, cols), jnp.float32),
            pltpu.VMEM((_CHUNK_ROWS, cols), jnp.float32),
            pltpu.VMEM((4 * _L,), jnp.float32),
            pltpu.VMEM((5 * _L,), jnp.float32),
            pltpu.VMEM((_L,), jnp.float32),
            pltpu.SemaphoreType.DMA,
            pltpu.SemaphoreType.DMA,
            pltpu.SemaphoreType.DMA,
            pltpu.SemaphoreType.DMA,
        ],
    )
    def wmse(pred_hbm, tgt_hbm, thr_hbm, dw_hbm, out_hbm,
             p0, p1, t0, t1, thr_v, dw_v, acc_v,
             sp0, sp1, st0, st1):
        wid = lax.axis_index("s") * nc + lax.axis_index("c")
        base = row_offset + wid * per_w
        pbufs, tbufs = (p0, p1), (t0, t1)
        psems, tsems = (sp0, sp1), (st0, st1)

        pltpu.sync_copy(thr_hbm, thr_v)
        pltpu.sync_copy(dw_hbm, dw_v)

        def dma_start(slot, g):
            row = base + g * _CHUNK_ROWS
            pltpu.async_copy(pred_hbm.at[pl.ds(row, _CHUNK_ROWS)],
                             pbufs[slot], psems[slot])
            pltpu.async_copy(tgt_hbm.at[pl.ds(row, _CHUNK_ROWS)],
                             tbufs[slot], tsems[slot])

        def dma_wait(slot):
            pltpu.make_async_copy(pred_hbm.at[pl.ds(0, _CHUNK_ROWS)],
                                  pbufs[slot], psems[slot]).wait()
            pltpu.make_async_copy(tgt_hbm.at[pl.ds(0, _CHUNK_ROWS)],
                                  tbufs[slot], tsems[slot]).wait()

        dma_start(0, 0)
        dma_start(1, 1)

        ths = [thr_v[pl.ds(k * _L, _L)] for k in range(4)]
        dws = [dw_v[pl.ds(k * _L, _L)] for k in range(5)]
        zero = jnp.zeros((_L,), jnp.float32)
        unroll = 4

        def chunk_acc(pbuf, tbuf, accs):
            def row_body(r, accs):
                for c in range(0, vregs_per_row, unroll):
                    ps = [pbuf[r, pl.ds((c + j) * _L, _L)]
                          for j in range(unroll)]
                    ts = [tbuf[r, pl.ds((c + j) * _L, _L)]
                          for j in range(unroll)]
                    d2s = [(p - t) * (p - t) for p, t in zip(ps, ts)]
                    new = []
                    for k in range(5):
                        if k == 0:
                            terms = d2s
                        else:
                            terms = [jnp.where(t >= ths[k - 1], d2, zero)
                                     for t, d2 in zip(ts, d2s)]
                        s01 = terms[0] + terms[1]
                        s23 = terms[2] + terms[3]
                        new.append(accs[k] + (s01 + s23))
                    accs = tuple(new)
                return accs
            return lax.fori_loop(0, _CHUNK_ROWS, row_body, accs)

        def outer(k, accs):
            for b in range(2):
                g = 2 * k + b
                dma_wait(b)
                accs = chunk_acc(pbufs[b], tbufs[b], accs)

                @pl.when(g + 2 < steps)
                def _():
                    dma_start(b, g + 2)
            return accs

        accs = lax.fori_loop(0, steps // 2, outer, (zero,) * 5)
        acc = dws[0] * accs[0]
        for k in range(1, 5):
            acc = acc + dws[k] * accs[k]
        acc_v[...] = acc
        pltpu.sync_copy(acc_v, out_hbm.at[wid])

    return wmse


_TC_BLOCK_ROWS = 512


def _tc_body(pred_ref, tgt_ref, thr_ref, dw_ref, out_ref, acc_ref):
    i = pl.program_id(0)

    @pl.when(i == 0)
    def _():
        acc_ref[...] = jnp.zeros_like(acc_ref)

    p = pred_ref[...]
    t = tgt_ref[...]
    d2 = (p - t) * (p - t)
    w = jnp.full_like(t, dw_ref[0])
    for k in range(4):
        w = w + jnp.where(t >= thr_ref[k], dw_ref[k + 1], 0.0)
    acc_ref[...] += w * d2

    @pl.when(i == pl.num_programs(0) - 1)
    def _():
        out_ref[0] = jnp.sum(acc_ref[...])


def _tc_wmse(pred2d, tgt2d, thr, dw, tc_rows):
    cols = pred2d.shape[1]
    grid = tc_rows // _TC_BLOCK_ROWS
    blk = pl.BlockSpec((_TC_BLOCK_ROWS, cols), lambda i: (i, 0))
    smem = pl.BlockSpec(memory_space=pltpu.SMEM)
    out = pl.pallas_call(
        _tc_body,
        grid=(grid,),
        in_specs=[blk, blk, smem, smem],
        out_specs=pl.BlockSpec(memory_space=pltpu.SMEM),
        out_shape=jax.ShapeDtypeStruct((1,), jnp.float32),
        scratch_shapes=[pltpu.VMEM((_TC_BLOCK_ROWS, cols), jnp.float32)],
    )(pred2d, tgt2d, thr, dw)
    return out[0]


_SC_ROWS = 10240


def kernel(prediction, target, weights, thresholds):
    n = prediction.size
    cols = prediction.shape[-1]
    rows = n // cols
    pred2d = prediction.reshape(rows, cols)
    tgt2d = target.reshape(rows, cols)
    dw = jnp.concatenate([weights[:1], jnp.diff(weights)])

    sc_rows = min(_SC_ROWS, rows)
    tc_rows = rows - sc_rows
    total = jnp.float32(0.0)
    if sc_rows > 0:
        thr_b = jnp.broadcast_to(thresholds[:, None], (4, _L)).reshape(4 * _L)
        dw_b = jnp.broadcast_to(dw[:, None], (5, _L)).reshape(5 * _L)
        sc_wmse = _make_sc_wmse(tc_rows, sc_rows, cols)
        partials = sc_wmse(pred2d, tgt2d, thr_b, dw_b)
        total = total + jnp.sum(partials)
    if tc_rows > 0:
        total = total + _tc_wmse(pred2d, tgt2d, thresholds, dw, tc_rows)
    return total / n


# speedup vs baseline: 12108.0024x; 1.0295x over previous
import functools

import jax
import jax.numpy as jnp
from jax import lax
from jax.experimental import pallas as pl
from jax.experimental.pallas import tpu as pltpu
from jax.experimental.pallas import tpu_sc as plsc

_L = 16
_CHUNK_ROWS = 16


def _make_sc_wmse(row_offset, sc_rows, cols):
    info = plsc.get_sparse_core_info()
    nc, ns = info.num_cores, info.num_subcores
    nw = nc * ns
    assert sc_rows % (nw * 2 * _CHUNK_ROWS) == 0
    per_w = sc_rows // nw
    steps = per_w // _CHUNK_ROWS
    vregs_per_row = cols // _L

    mesh = plsc.VectorSubcoreMesh(core_axis_name="c", subcore_axis_name="s")

    @functools.partial(
        pl.kernel,
        mesh=mesh,
        out_type=jax.ShapeDtypeStruct((nw, _L), jnp.float32),
        scratch_types=[
            pltpu.VMEM((_CHUNK_ROWS, cols), jnp.float32),
            pltpu.VMEM((_CHUNK_ROWS, cols), jnp.float32),
            pltpu.VMEM((_CHUNK_ROWS, cols), jnp.float32),
            pltpu.VMEM((_CHUNK_ROWS, cols), jnp.float32),
            pltpu.VMEM((4 * _L,), jnp.float32),
            pltpu.VMEM((5 * _L,), jnp.float32),
            pltpu.VMEM((_L,), jnp.float32),
            pltpu.SemaphoreType.DMA,
            pltpu.SemaphoreType.DMA,
            pltpu.SemaphoreType.DMA,
            pltpu.SemaphoreType.DMA,
        ],
    )
    def wmse(pred_hbm, tgt_hbm, thr_hbm, dw_hbm, out_hbm,
             p0, p1, t0, t1, thr_v, dw_v, acc_v,
             sp0, sp1, st0, st1):
        wid = lax.axis_index("s") * nc + lax.axis_index("c")
        base = row_offset + wid * per_w
        pbufs, tbufs = (p0, p1), (t0, t1)
        psems, tsems = (sp0, sp1), (st0, st1)

        pltpu.sync_copy(thr_hbm, thr_v)
        pltpu.sync_copy(dw_hbm, dw_v)

        def dma_start(slot, g):
            row = base + g * _CHUNK_ROWS
            pltpu.async_copy(pred_hbm.at[pl.ds(row, _CHUNK_ROWS)],
                             pbufs[slot], psems[slot])
            pltpu.async_copy(tgt_hbm.at[pl.ds(row, _CHUNK_ROWS)],
                             tbufs[slot], tsems[slot])

        def dma_wait(slot):
            pltpu.make_async_copy(pred_hbm.at[pl.ds(0, _CHUNK_ROWS)],
                                  pbufs[slot], psems[slot]).wait()
            pltpu.make_async_copy(tgt_hbm.at[pl.ds(0, _CHUNK_ROWS)],
                                  tbufs[slot], tsems[slot]).wait()

        dma_start(0, 0)
        dma_start(1, 1)

        ths = [thr_v[pl.ds(k * _L, _L)] for k in range(4)]
        dws = [dw_v[pl.ds(k * _L, _L)] for k in range(5)]
        zero = jnp.zeros((_L,), jnp.float32)
        unroll = 4

        def chunk_acc(pbuf, tbuf, accs):
            def row_body(r, accs):
                for c in range(0, vregs_per_row, unroll):
                    ps = [pbuf[r, pl.ds((c + j) * _L, _L)]
                          for j in range(unroll)]
                    ts = [tbuf[r, pl.ds((c + j) * _L, _L)]
                          for j in range(unroll)]
                    d2s = [(p - t) * (p - t) for p, t in zip(ps, ts)]
                    new = []
                    for k in range(5):
                        if k == 0:
                            terms = d2s
                        else:
                            terms = [jnp.where(t >= ths[k - 1], d2, zero)
                                     for t, d2 in zip(ts, d2s)]
                        s01 = terms[0] + terms[1]
                        s23 = terms[2] + terms[3]
                        new.append(accs[k] + (s01 + s23))
                    accs = tuple(new)
                return accs
            return lax.fori_loop(0, _CHUNK_ROWS, row_body, accs)

        def outer(k, accs):
            for b in range(2):
                g = 2 * k + b
                dma_wait(b)
                accs = chunk_acc(pbufs[b], tbufs[b], accs)

                @pl.when(g + 2 < steps)
                def _():
                    dma_start(b, g + 2)
            return accs

        accs = lax.fori_loop(0, steps // 2, outer, (zero,) * 5)
        acc = dws[0] * accs[0]
        for k in range(1, 5):
            acc = acc + dws[k] * accs[k]
        acc_v[...] = acc
        pltpu.sync_copy(acc_v, out_hbm.at[wid])

    return wmse


_TC_BLOCK_ROWS = 512


def _tc_body(pred_ref, tgt_ref, thr_ref, dw_ref, out_ref, acc_ref):
    i = pl.program_id(0)

    @pl.when(i == 0)
    def _():
        acc_ref[...] = jnp.zeros_like(acc_ref)

    p = pred_ref[...]
    t = tgt_ref[...]
    d2 = (p - t) * (p - t)
    w = jnp.full_like(t, dw_ref[0])
    for k in range(4):
        w = w + jnp.where(t >= thr_ref[k], dw_ref[k + 1], 0.0)
    acc_ref[...] += w * d2

    @pl.when(i == pl.num_programs(0) - 1)
    def _():
        out_ref[0] = jnp.sum(acc_ref[...])


def _tc_wmse(pred2d, tgt2d, thr, dw, tc_rows):
    cols = pred2d.shape[1]
    grid = tc_rows // _TC_BLOCK_ROWS
    blk = pl.BlockSpec((_TC_BLOCK_ROWS, cols), lambda i: (i, 0))
    smem = pl.BlockSpec(memory_space=pltpu.SMEM)
    out = pl.pallas_call(
        _tc_body,
        grid=(grid,),
        in_specs=[blk, blk, smem, smem],
        out_specs=pl.BlockSpec(memory_space=pltpu.SMEM),
        out_shape=jax.ShapeDtypeStruct((1,), jnp.float32),
        scratch_shapes=[pltpu.VMEM((_TC_BLOCK_ROWS, cols), jnp.float32)],
    )(pred2d, tgt2d, thr, dw)
    return out[0]


_SC_ROWS = 11264


def kernel(prediction, target, weights, thresholds):
    n = prediction.size
    cols = prediction.shape[-1]
    rows = n // cols
    pred2d = prediction.reshape(rows, cols)
    tgt2d = target.reshape(rows, cols)
    dw = jnp.concatenate([weights[:1], jnp.diff(weights)])

    sc_rows = min(_SC_ROWS, rows)
    tc_rows = rows - sc_rows
    total = jnp.float32(0.0)
    if sc_rows > 0:
        thr_b = jnp.broadcast_to(thresholds[:, None], (4, _L)).reshape(4 * _L)
        dw_b = jnp.broadcast_to(dw[:, None], (5, _L)).reshape(5 * _L)
        sc_wmse = _make_sc_wmse(tc_rows, sc_rows, cols)
        partials = sc_wmse(pred2d, tgt2d, thr_b, dw_b)
        total = total + jnp.sum(partials)
    if tc_rows > 0:
        total = total + _tc_wmse(pred2d, tgt2d, thresholds, dw, tc_rows)
    return total / n
